# baseline (speedup 1.0000x reference)
import functools

import jax
import jax.numpy as jnp
from jax import lax
from jax.experimental import pallas as pl
from jax.experimental.pallas import tpu as pltpu
from jax.experimental.pallas import tpu_sc as plsc

VOCAB = 1000000
D_MODEL = 64
BATCH = 4096
SEQ = 200

_INFO = plsc.get_sparse_core_info()
_NC = _INFO.num_cores
_NS = _INFO.num_subcores
_NW = _NC * _NS

_N = BATCH * SEQ
_PER_W = _N // _NW
_CHUNK = 512
_ITERS = _PER_W // _CHUNK


def _make_gather():
    mesh = plsc.VectorSubcoreMesh(core_axis_name="c", subcore_axis_name="s")

    @functools.partial(
        pl.kernel,
        mesh=mesh,
        out_type=jax.ShapeDtypeStruct((_N, D_MODEL), jnp.float32),
        scratch_types=[
            pltpu.VMEM((_CHUNK,), jnp.int32),
            pltpu.VMEM((_CHUNK, D_MODEL), jnp.float32),
            pltpu.SemaphoreType.DMA,
        ],
        compiler_params=pltpu.CompilerParams(use_tc_tiling_on_sc=False),
    )
    def gather_kernel(idx_hbm, table_hbm, out_hbm, idx_v, rows_v, sem):
        wid = lax.axis_index("s") * _NC + lax.axis_index("c")
        w_base = wid * _PER_W

        def step(i, carry):
            base = w_base + i * _CHUNK
            pltpu.sync_copy(idx_hbm.at[pl.ds(base, _CHUNK)], idx_v)
            pltpu.async_copy(table_hbm.at[idx_v], rows_v, sem).wait()
            pltpu.sync_copy(rows_v, out_hbm.at[pl.ds(base, _CHUNK)])
            return carry

        lax.fori_loop(0, _ITERS, step, 0)

    return gather_kernel


_gather = _make_gather()


def kernel(inp, table):
    idx = jnp.transpose(inp).reshape(_N).astype(jnp.int32)
    out = _gather(idx, table)
    return out.reshape(SEQ, BATCH, D_MODEL)


# speedup vs baseline: 1.0481x; 1.0481x over previous
import functools

import jax
import jax.numpy as jnp
from jax import lax
from jax.experimental import pallas as pl
from jax.experimental.pallas import tpu as pltpu
from jax.experimental.pallas import tpu_sc as plsc

VOCAB = 1000000
D_MODEL = 64
BATCH = 4096
SEQ = 200

_INFO = plsc.get_sparse_core_info()
_NC = _INFO.num_cores
_NS = _INFO.num_subcores
_NW = _NC * _NS

_N = BATCH * SEQ
_PER_W = _N // _NW
_CHUNK = 800
_ITERS = _PER_W // _CHUNK
_HALF = _ITERS // 2


def _make_gather():
    mesh = plsc.VectorSubcoreMesh(core_axis_name="c", subcore_axis_name="s")

    @functools.partial(
        pl.kernel,
        mesh=mesh,
        out_type=jax.ShapeDtypeStruct((_N, D_MODEL), jnp.float32),
        scratch_types=[
            pltpu.VMEM((_ITERS, _CHUNK), jnp.int32),
            pltpu.VMEM((_CHUNK, D_MODEL), jnp.float32),
            pltpu.VMEM((_CHUNK, D_MODEL), jnp.float32),
            pltpu.SemaphoreType.DMA,
            pltpu.SemaphoreType.DMA,
        ],
        compiler_params=pltpu.CompilerParams(use_tc_tiling_on_sc=False),
    )
    def gather_kernel(idx_hbm, table_hbm, out_hbm, idx_v, rows0, rows1,
                      sg0, sg1):
        wid = lax.axis_index("s") * _NC + lax.axis_index("c")
        w_base = wid * _PER_W

        pltpu.sync_copy(idx_hbm.at[pl.ds(wid * _ITERS, _ITERS)], idx_v)

        def g_start(i, rows, sem):
            pltpu.async_copy(table_hbm.at[idx_v.at[i]], rows, sem)

        def g_wait(i, rows, sem):
            pltpu.make_async_copy(table_hbm.at[idx_v.at[i]], rows, sem).wait()

        def write(i, rows):
            pltpu.sync_copy(rows, out_hbm.at[pl.ds(w_base + i * _CHUNK, _CHUNK)])

        g_start(0, rows0, sg0)

        def step(j, carry):
            i0 = 2 * j
            g_start(i0 + 1, rows1, sg1)
            g_wait(i0, rows0, sg0)
            write(i0, rows0)

            @pl.when(j + 1 < _HALF)
            def _():
                g_start(i0 + 2, rows0, sg0)

            g_wait(i0 + 1, rows1, sg1)
            write(i0 + 1, rows1)
            return carry

        lax.fori_loop(0, _HALF, step, 0)

    return gather_kernel


_gather = _make_gather()


def kernel(inp, table):
    idx = jnp.transpose(inp).reshape(_NW * _ITERS, _CHUNK).astype(jnp.int32)
    out = _gather(idx, table)
    return out.reshape(SEQ, BATCH, D_MODEL)


# speedup vs baseline: 1.0511x; 1.0029x over previous
import functools

import jax
import jax.numpy as jnp
from jax import lax
from jax.experimental import pallas as pl
from jax.experimental.pallas import tpu as pltpu
from jax.experimental.pallas import tpu_sc as plsc

VOCAB = 1000000
D_MODEL = 64
BATCH = 4096
SEQ = 200

_INFO = plsc.get_sparse_core_info()
_NC = _INFO.num_cores
_NS = _INFO.num_subcores
_NW = _NC * _NS

_N = BATCH * SEQ
_PER_W = _N // _NW
_NB = 4
_CHUNK = 400
_ITERS = _PER_W // _CHUNK


def _make_gather():
    mesh = plsc.VectorSubcoreMesh(core_axis_name="c", subcore_axis_name="s")

    @functools.partial(
        pl.kernel,
        mesh=mesh,
        out_type=jax.ShapeDtypeStruct((_N, D_MODEL), jnp.float32),
        scratch_types=(
            [pltpu.VMEM((_ITERS, _CHUNK), jnp.int32)]
            + [pltpu.VMEM((_CHUNK, D_MODEL), jnp.float32) for _ in range(_NB)]
            + [pltpu.SemaphoreType.DMA for _ in range(2 * _NB)]
        ),
        compiler_params=pltpu.CompilerParams(use_tc_tiling_on_sc=False),
    )
    def gather_kernel(idx_hbm, table_hbm, out_hbm, idx_v, *rest):
        rows = rest[:_NB]
        sg = rest[_NB:2 * _NB]
        sw = rest[2 * _NB:]
        wid = lax.axis_index("s") * _NC + lax.axis_index("c")
        w_base = wid * _PER_W

        pltpu.sync_copy(idx_hbm.at[pl.ds(wid * _ITERS, _ITERS)], idx_v)

        def g_start(i, b):
            pltpu.async_copy(table_hbm.at[idx_v.at[i]], rows[b], sg[b])

        def g_wait(i, b):
            pltpu.make_async_copy(table_hbm.at[idx_v.at[i]], rows[b],
                                  sg[b]).wait()

        def out_slc(i):
            return out_hbm.at[pl.ds(w_base + i * _CHUNK, _CHUNK)]

        def w_start(i, b):
            pltpu.async_copy(rows[b], out_slc(i), sw[b])

        def w_wait(i, b):
            pltpu.make_async_copy(rows[b], out_slc(i), sw[b]).wait()

        for b in range(_NB):
            g_start(b, b)

        def step(j, carry):
            base = j * _NB
            for b in range(_NB):
                i = base + b
                g_wait(i, b)
                w_start(i, b)
                pb = (b - 1) % _NB
                pi = i + _NB - 1

                @pl.when(jnp.logical_and(pi >= _NB, pi < _ITERS))
                def _():
                    w_wait(pi - _NB, pb)
                    g_start(pi, pb)
            return carry

        lax.fori_loop(0, _ITERS // _NB, step, 0)

        for b in range(_NB):
            w_wait(_ITERS - _NB + b, b)

    return gather_kernel


_gather = _make_gather()


def kernel(inp, table):
    idx = jnp.transpose(inp).reshape(_NW * _ITERS, _CHUNK).astype(jnp.int32)
    out = _gather(idx, table)
    return out.reshape(SEQ, BATCH, D_MODEL)
